# baseline (speedup 1.0000x reference)
import functools

import jax
import jax.numpy as jnp
from jax import lax
from jax.experimental import pallas as pl
from jax.experimental.pallas import tpu as pltpu
from jax.experimental.pallas import tpu_sc as plsc

R = 128
N = 32768
K = 64
L = 16
NV = N // L
GV = 8
NG = NV // GV
NSEG = 4
SEGG = NG // NSEG

_info = plsc.get_sparse_core_info()
NC, NS = _info.num_cores, _info.num_subcores
NW = NC * NS
RPW = R // NW

_LANE = lambda: lax.iota(jnp.int32, L)


def _bmax(v):
    for s in (8, 4, 2, 1):
        v = jnp.maximum(v, v[_LANE() ^ s])
    return v


def _bmin(v):
    for s in (8, 4, 2, 1):
        v = jnp.minimum(v, v[_LANE() ^ s])
    return v


def _bsum(v):
    for s in (8, 4, 2, 1):
        v = v + v[_LANE() ^ s]
    return v


def _process_row(row_v, cand_v):
    seg_maxes = []
    for s in range(NSEG):
        def gbody(g, m_seg, s=s):
            gbase = s * SEGG + g
            vs = [row_v[pl.ds((gbase * GV + j) * L, L)] for j in range(GV)]
            m01 = jnp.maximum(vs[0], vs[1])
            m23 = jnp.maximum(vs[2], vs[3])
            m45 = jnp.maximum(vs[4], vs[5])
            m67 = jnp.maximum(vs[6], vs[7])
            m_grp = jnp.maximum(jnp.maximum(m01, m23), jnp.maximum(m45, m67))
            return jnp.maximum(m_seg, m_grp)
        seg_maxes.append(
            lax.fori_loop(0, SEGG, gbody, jnp.zeros((L,), jnp.float32)))
    t01 = jnp.minimum(seg_maxes[0], seg_maxes[1])
    t23 = jnp.minimum(seg_maxes[2], seg_maxes[3])
    t_hat = _bmin(jnp.minimum(t01, t23))[0]
    row_max = _bmax(jnp.maximum(jnp.maximum(seg_maxes[0], seg_maxes[1]),
                                jnp.maximum(seg_maxes[2], seg_maxes[3])))[0]

    t_spl = jnp.zeros((L,), jnp.float32) + t_hat
    BV = 2 * L
    NB = NV // BV
    def block_body(b, off):
        w_lo = jnp.zeros((L,), jnp.int32)
        w_hi = jnp.zeros((L,), jnp.int32)
        for i in range(L):
            v = row_v[pl.ds((b * BV + i) * L, L)]
            w_lo = w_lo | jnp.where(v >= t_spl, jnp.int32(1 << i), 0)
        for i in range(L):
            v = row_v[pl.ds((b * BV + L + i) * L, L)]
            w_hi = w_hi | jnp.where(v >= t_spl, jnp.int32(1 << i), 0)
        w = w_lo | lax.shift_left(w_hi, 16)
        for s in (8, 4, 2, 1):
            w = w | w[_LANE() ^ s]
        m = w[0]
        pc = m - (lax.shift_right_logical(m, 1) & 0x55555555)
        pc = (pc & 0x33333333) + (lax.shift_right_logical(pc, 2) & 0x33333333)
        pc = (pc + lax.shift_right_logical(pc, 4)) & 0x0F0F0F0F
        pc = pc + lax.shift_right_logical(pc, 8)
        pc = (pc + lax.shift_right_logical(pc, 16)) & 0x3F
        def take(j, carry):
            mm, o = carry
            hi16 = lax.shift_right_logical(mm, 16)
            use_hi = hi16 > 0
            val = jnp.where(use_hi, hi16, mm & 0xFFFF)
            e = (jnp.where(use_hi, 16, 0)
                 + lax.shift_right_logical(
                     lax.bitcast_convert_type(val.astype(jnp.float32),
                                              jnp.int32), 23) - 127)
            cand_v[pl.ds(o, L)] = row_v[pl.ds(b * (BV * L) + e * L, L)]
            return (mm - lax.shift_left(jnp.int32(1), e), o + L)
        _, off = lax.fori_loop(0, pc, take, (m, off))
        return off
    off = lax.fori_loop(0, NB, block_body, jnp.int32(0))

    zero_v = jnp.zeros((L,), jnp.float32)
    for j in range(7):
        cand_v[pl.ds(off + j * L, L)] = zero_v
    nv8 = lax.shift_right_logical(off + 7 * L, 7)

    def count_gt(tf):
        tf_spl = jnp.zeros((L,), jnp.float32) + tf
        def cbody(i, accs):
            a0, a1, a2, a3 = accs
            b = i * (8 * L)
            vs = [cand_v[pl.ds(b + j * L, L)] for j in range(8)]
            a0 = a0 + jnp.where(vs[0] > tf_spl, 1, 0) + jnp.where(vs[4] > tf_spl, 1, 0)
            a1 = a1 + jnp.where(vs[1] > tf_spl, 1, 0) + jnp.where(vs[5] > tf_spl, 1, 0)
            a2 = a2 + jnp.where(vs[2] > tf_spl, 1, 0) + jnp.where(vs[6] > tf_spl, 1, 0)
            a3 = a3 + jnp.where(vs[3] > tf_spl, 1, 0) + jnp.where(vs[7] > tf_spl, 1, 0)
            return (a0, a1, a2, a3)
        z = jnp.zeros((L,), jnp.int32)
        a0, a1, a2, a3 = lax.fori_loop(0, nv8, cbody, (z, z, z, z))
        return _bsum((a0 + a1) + (a2 + a3))[0]

    lo0 = lax.bitcast_convert_type(t_hat, jnp.int32) - 1
    hi0 = lax.bitcast_convert_type(row_max, jnp.int32) + 1
    cnt_lo0 = count_gt(lax.bitcast_convert_type(
        jnp.maximum(lo0, 0), jnp.float32))

    def bbody(j, state):
        def probe(state):
            lo, hi, cnt_lo, cnt_hi = state
            lo_f = lax.bitcast_convert_type(jnp.maximum(lo, 0), jnp.float32)
            hi_f = lax.bitcast_convert_type(hi, jnp.float32)
            d_f = (cnt_lo - cnt_hi).astype(jnp.float32)
            r = lax.bitcast_convert_type(
                jnp.int32(0x7EF311C3)
                - lax.bitcast_convert_type(d_f, jnp.int32), jnp.float32)
            r = r * (jnp.float32(2.0) - d_f * r)
            t_next = lo_f + ((cnt_lo - K).astype(jnp.float32)
                             * r * (hi_f - lo_f))
            mid_interp = lax.bitcast_convert_type(t_next, jnp.int32)
            mid_bisect = lo + lax.shift_right_logical(hi - lo, 1)
            mid = jnp.where((j & 1) == 0, mid_interp, mid_bisect)
            mid = jnp.minimum(jnp.maximum(mid, lo + 1), hi - 1)
            mid = jnp.maximum(mid, jnp.int32(0))
            cnt = count_gt(lax.bitcast_convert_type(mid, jnp.float32))
            big = cnt >= K
            return (jnp.where(big, mid, lo), jnp.where(big, hi, mid),
                    jnp.where(big, cnt, cnt_lo), jnp.where(big, cnt_hi, cnt))
        lo, hi, _, _ = state
        return lax.cond(hi - lo > 1, probe, lambda s: s, state)
    _, kth_bits, _, _ = lax.fori_loop(
        0, 64, bbody, (lo0, hi0, cnt_lo0, jnp.int32(0)))
    kth_f = lax.bitcast_convert_type(kth_bits, jnp.float32)

    kth_spl = jnp.zeros((L,), jnp.float32) + kth_f
    def fbody(i, carry):
        s0, s1, c0, c1 = carry
        b = i * (8 * L)
        vs = [cand_v[pl.ds(b + j * L, L)] for j in range(8)]
        ms = [v > kth_spl for v in vs]
        s0 = s0 + jnp.where(ms[0], vs[0], 0.0) + jnp.where(ms[4], vs[4], 0.0)
        s0 = s0 + jnp.where(ms[1], vs[1], 0.0) + jnp.where(ms[5], vs[5], 0.0)
        s1 = s1 + jnp.where(ms[2], vs[2], 0.0) + jnp.where(ms[6], vs[6], 0.0)
        s1 = s1 + jnp.where(ms[3], vs[3], 0.0) + jnp.where(ms[7], vs[7], 0.0)
        c0 = c0 + jnp.where(ms[0], 1, 0) + jnp.where(ms[4], 1, 0)
        c0 = c0 + jnp.where(ms[1], 1, 0) + jnp.where(ms[5], 1, 0)
        c1 = c1 + jnp.where(ms[2], 1, 0) + jnp.where(ms[6], 1, 0)
        c1 = c1 + jnp.where(ms[3], 1, 0) + jnp.where(ms[7], 1, 0)
        return (s0, s1, c0, c1)
    zf = jnp.zeros((L,), jnp.float32)
    zi = jnp.zeros((L,), jnp.int32)
    s0, s1, c0, c1 = lax.fori_loop(0, nv8, fbody, (zf, zf, zi, zi))
    cnt_gt = _bsum(c0 + c1)[0]
    return (_bsum(s0 + s1)[0]
            + (jnp.float32(K) - cnt_gt.astype(jnp.float32)) * kth_f)


def _topk_sums_body(x_hbm, out_hbm, row_a, row_b, cand_v, out_v,
                    sem_a, sem_b):
    wid = lax.axis_index("s") * NC + lax.axis_index("c")
    base = wid * RPW
    out_acc = jnp.zeros((L,), jnp.float32)

    bufs = (row_a, row_b)
    sems = (sem_a, sem_b)
    h = pltpu.async_copy(x_hbm.at[base], row_a, sem_a)
    for r in range(RPW):
        h.wait()
        if r + 1 < RPW:
            h = pltpu.async_copy(x_hbm.at[base + r + 1],
                                 bufs[(r + 1) % 2], sems[(r + 1) % 2])
        ssum = _process_row(bufs[r % 2], cand_v)
        out_acc = jnp.where(_LANE() == r, ssum, out_acc)

    out_v[...] = out_acc
    pltpu.sync_copy(out_v, out_hbm.at[wid])


_topk_sums = functools.partial(
    pl.kernel,
    out_type=jax.ShapeDtypeStruct((NW, L), jnp.float32),
    mesh=plsc.VectorSubcoreMesh(core_axis_name="c", subcore_axis_name="s"),
    scratch_types=[
        pltpu.VMEM((N,), jnp.float32),
        pltpu.VMEM((N,), jnp.float32),
        pltpu.VMEM((N + 8 * L,), jnp.float32),
        pltpu.VMEM((L,), jnp.float32),
        pltpu.SemaphoreType.DMA,
        pltpu.SemaphoreType.DMA,
    ],
)(_topk_sums_body)


def _loss_body(s_ref, o_ref):
    s = s_ref[...]
    col = lax.broadcasted_iota(jnp.int32, (NW, L), 1)
    term = jnp.where(col < RPW, -jnp.log(s), 0.0)
    o_ref[0, 0] = jnp.sum(term) * jnp.float32(1.0 / R)


_loss_call = pl.pallas_call(
    _loss_body,
    out_shape=jax.ShapeDtypeStruct((1, 1), jnp.float32),
    out_specs=pl.BlockSpec(memory_space=pltpu.SMEM),
)


@jax.jit
def kernel(inputs, k):
    sums = _topk_sums(inputs)
    loss = _loss_call(sums)[0, 0]
    return loss + jnp.asarray(k, jnp.float32) * 0.0


# speedup vs baseline: 1.5243x; 1.5243x over previous
import functools

import jax
import jax.numpy as jnp
from jax import lax
from jax.experimental import pallas as pl
from jax.experimental.pallas import tpu as pltpu
from jax.experimental.pallas import tpu_sc as plsc

R = 128
N = 32768
K = 64
L = 16
NV = N // L
GV = 8
NG = NV // GV
NSEG = 4
SEGG = NG // NSEG

_info = plsc.get_sparse_core_info()
NC, NS = _info.num_cores, _info.num_subcores
NW = NC * NS
RPW = R // NW

_LANE = lambda: lax.iota(jnp.int32, L)


def _bmax(v):
    for s in (8, 4, 2, 1):
        v = jnp.maximum(v, v[_LANE() ^ s])
    return v


def _bmin(v):
    for s in (8, 4, 2, 1):
        v = jnp.minimum(v, v[_LANE() ^ s])
    return v


def _bsum(v):
    for s in (8, 4, 2, 1):
        v = v + v[_LANE() ^ s]
    return v


def _process_row(row_v, cand_v):
    def _merge2(a, b):
        a1, a2 = a
        b1, b2 = b
        return (jnp.maximum(a1, b1),
                jnp.maximum(jnp.minimum(a1, b1), jnp.maximum(a2, b2)))
    samples = []
    for s in range(NSEG):
        def gbody(g, m_seg, s=s):
            gbase = s * SEGG + g
            vs = [row_v[pl.ds((gbase * GV + j) * L, L)] for j in range(GV)]
            p = [(jnp.maximum(vs[2 * i], vs[2 * i + 1]),
                  jnp.minimum(vs[2 * i], vs[2 * i + 1])) for i in range(4)]
            q = (_merge2(p[0], p[1]), _merge2(p[2], p[3]))
            grp = _merge2(q[0], q[1])
            return _merge2(m_seg, grp)
        neg = jnp.full((L,), -jnp.inf, jnp.float32)
        samples.extend(lax.fori_loop(0, SEGG, gbody, (neg, neg)))
    row_max = _bmax(jnp.maximum(jnp.maximum(samples[0], samples[2]),
                                jnp.maximum(samples[4], samples[6])))[0]
    s_min = _bmin(jnp.minimum(jnp.minimum(samples[1], samples[3]),
                              jnp.minimum(samples[5], samples[7])))[0]

    def count_samples_gt(tf):
        tf_spl = jnp.zeros((L,), jnp.float32) + tf
        acc = jnp.zeros((L,), jnp.int32)
        for sv in samples:
            acc = acc + jnp.where(sv > tf_spl, 1, 0)
        return _bsum(acc)[0]

    slo0 = lax.bitcast_convert_type(s_min, jnp.int32) - 1
    shi0 = lax.bitcast_convert_type(row_max, jnp.int32) + 1
    srng = (shi0 - slo0).astype(jnp.float32)
    siters = (lax.shift_right_logical(
        lax.bitcast_convert_type(srng, jnp.int32), 23) & 0xFF) - 126

    def sbody(j, lohi):
        lo, hi = lohi
        mid = jnp.maximum(lo + lax.shift_right_logical(hi - lo, 1),
                          jnp.int32(0))
        midf = lax.bitcast_convert_type(mid, jnp.float32)
        big = count_samples_gt(midf) >= K
        return (jnp.where(big, mid, lo), jnp.where(big, hi, mid))
    _, that_bits = lax.fori_loop(0, siters, sbody, (slo0, shi0))
    t_hat = lax.bitcast_convert_type(that_bits, jnp.float32)

    t_spl = jnp.zeros((L,), jnp.float32) + t_hat
    BV = 2 * L
    NB = NV // BV
    def block_body(b, off):
        w_lo = jnp.zeros((L,), jnp.int32)
        w_hi = jnp.zeros((L,), jnp.int32)
        for i in range(L):
            v = row_v[pl.ds((b * BV + i) * L, L)]
            w_lo = w_lo | jnp.where(v >= t_spl, jnp.int32(1 << i), 0)
        for i in range(L):
            v = row_v[pl.ds((b * BV + L + i) * L, L)]
            w_hi = w_hi | jnp.where(v >= t_spl, jnp.int32(1 << i), 0)
        w = w_lo | lax.shift_left(w_hi, 16)
        for s in (8, 4, 2, 1):
            w = w | w[_LANE() ^ s]
        m = w[0]
        pc = m - (lax.shift_right_logical(m, 1) & 0x55555555)
        pc = (pc & 0x33333333) + (lax.shift_right_logical(pc, 2) & 0x33333333)
        pc = (pc + lax.shift_right_logical(pc, 4)) & 0x0F0F0F0F
        pc = pc + lax.shift_right_logical(pc, 8)
        pc = (pc + lax.shift_right_logical(pc, 16)) & 0x3F
        def take(j, carry):
            mm, o = carry
            hi16 = lax.shift_right_logical(mm, 16)
            use_hi = hi16 > 0
            val = jnp.where(use_hi, hi16, mm & 0xFFFF)
            e = (jnp.where(use_hi, 16, 0)
                 + lax.shift_right_logical(
                     lax.bitcast_convert_type(val.astype(jnp.float32),
                                              jnp.int32), 23) - 127)
            cand_v[pl.ds(o, L)] = row_v[pl.ds(b * (BV * L) + e * L, L)]
            return (mm - lax.shift_left(jnp.int32(1), e), o + L)
        _, off = lax.fori_loop(0, pc, take, (m, off))
        return off
    off = lax.fori_loop(0, NB, block_body, jnp.int32(0))

    zero_v = jnp.zeros((L,), jnp.float32)
    for j in range(7):
        cand_v[pl.ds(off + j * L, L)] = zero_v
    nv8 = lax.shift_right_logical(off + 7 * L, 7)

    def count_gt(tf):
        tf_spl = jnp.zeros((L,), jnp.float32) + tf
        def cbody(i, accs):
            a0, a1, a2, a3 = accs
            b = i * (8 * L)
            vs = [cand_v[pl.ds(b + j * L, L)] for j in range(8)]
            a0 = a0 + jnp.where(vs[0] > tf_spl, 1, 0) + jnp.where(vs[4] > tf_spl, 1, 0)
            a1 = a1 + jnp.where(vs[1] > tf_spl, 1, 0) + jnp.where(vs[5] > tf_spl, 1, 0)
            a2 = a2 + jnp.where(vs[2] > tf_spl, 1, 0) + jnp.where(vs[6] > tf_spl, 1, 0)
            a3 = a3 + jnp.where(vs[3] > tf_spl, 1, 0) + jnp.where(vs[7] > tf_spl, 1, 0)
            return (a0, a1, a2, a3)
        z = jnp.zeros((L,), jnp.int32)
        a0, a1, a2, a3 = lax.fori_loop(0, nv8, cbody, (z, z, z, z))
        return _bsum((a0 + a1) + (a2 + a3))[0]

    lo0 = lax.bitcast_convert_type(t_hat, jnp.int32) - 1
    hi0 = lax.bitcast_convert_type(row_max, jnp.int32) + 1
    rng_f = (hi0 - lo0).astype(jnp.float32)
    iters = (lax.shift_right_logical(
        lax.bitcast_convert_type(rng_f, jnp.int32), 23) & 0xFF) - 126

    def bbody(j, lohi):
        lo, hi = lohi
        mid = jnp.maximum(lo + lax.shift_right_logical(hi - lo, 1),
                          jnp.int32(0))
        midf = lax.bitcast_convert_type(mid, jnp.float32)
        big = count_gt(midf) >= K
        return (jnp.where(big, mid, lo), jnp.where(big, hi, mid))
    _, kth_bits = lax.fori_loop(0, iters, bbody, (lo0, hi0))
    kth_f = lax.bitcast_convert_type(kth_bits, jnp.float32)

    kth_spl = jnp.zeros((L,), jnp.float32) + kth_f
    def fbody(i, carry):
        s0, s1, c0, c1 = carry
        b = i * (8 * L)
        vs = [cand_v[pl.ds(b + j * L, L)] for j in range(8)]
        ms = [v > kth_spl for v in vs]
        s0 = s0 + jnp.where(ms[0], vs[0], 0.0) + jnp.where(ms[4], vs[4], 0.0)
        s0 = s0 + jnp.where(ms[1], vs[1], 0.0) + jnp.where(ms[5], vs[5], 0.0)
        s1 = s1 + jnp.where(ms[2], vs[2], 0.0) + jnp.where(ms[6], vs[6], 0.0)
        s1 = s1 + jnp.where(ms[3], vs[3], 0.0) + jnp.where(ms[7], vs[7], 0.0)
        c0 = c0 + jnp.where(ms[0], 1, 0) + jnp.where(ms[4], 1, 0)
        c0 = c0 + jnp.where(ms[1], 1, 0) + jnp.where(ms[5], 1, 0)
        c1 = c1 + jnp.where(ms[2], 1, 0) + jnp.where(ms[6], 1, 0)
        c1 = c1 + jnp.where(ms[3], 1, 0) + jnp.where(ms[7], 1, 0)
        return (s0, s1, c0, c1)
    zf = jnp.zeros((L,), jnp.float32)
    zi = jnp.zeros((L,), jnp.int32)
    s0, s1, c0, c1 = lax.fori_loop(0, nv8, fbody, (zf, zf, zi, zi))
    cnt_gt = _bsum(c0 + c1)[0]
    return (_bsum(s0 + s1)[0]
            + (jnp.float32(K) - cnt_gt.astype(jnp.float32)) * kth_f)


def _topk_sums_body(x_hbm, out_hbm, row_a, row_b, cand_v, out_v,
                    sem_a, sem_b):
    wid = lax.axis_index("s") * NC + lax.axis_index("c")
    base = wid * RPW
    out_acc = jnp.zeros((L,), jnp.float32)

    bufs = (row_a, row_b)
    sems = (sem_a, sem_b)
    h = pltpu.async_copy(x_hbm.at[base], row_a, sem_a)
    for r in range(RPW):
        h.wait()
        if r + 1 < RPW:
            h = pltpu.async_copy(x_hbm.at[base + r + 1],
                                 bufs[(r + 1) % 2], sems[(r + 1) % 2])
        ssum = _process_row(bufs[r % 2], cand_v)
        out_acc = jnp.where(_LANE() == r, ssum, out_acc)

    out_v[...] = out_acc
    pltpu.sync_copy(out_v, out_hbm.at[wid])


_topk_sums = functools.partial(
    pl.kernel,
    out_type=jax.ShapeDtypeStruct((NW, L), jnp.float32),
    mesh=plsc.VectorSubcoreMesh(core_axis_name="c", subcore_axis_name="s"),
    scratch_types=[
        pltpu.VMEM((N,), jnp.float32),
        pltpu.VMEM((N,), jnp.float32),
        pltpu.VMEM((N + 8 * L,), jnp.float32),
        pltpu.VMEM((L,), jnp.float32),
        pltpu.SemaphoreType.DMA,
        pltpu.SemaphoreType.DMA,
    ],
)(_topk_sums_body)


def _loss_body(s_ref, o_ref):
    s = s_ref[...]
    col = lax.broadcasted_iota(jnp.int32, (NW, L), 1)
    term = jnp.where(col < RPW, -jnp.log(s), 0.0)
    o_ref[0, 0] = jnp.sum(term) * jnp.float32(1.0 / R)


_loss_call = pl.pallas_call(
    _loss_body,
    out_shape=jax.ShapeDtypeStruct((1, 1), jnp.float32),
    out_specs=pl.BlockSpec(memory_space=pltpu.SMEM),
)


@jax.jit
def kernel(inputs, k):
    sums = _topk_sums(inputs)
    loss = _loss_call(sums)[0, 0]
    return loss + jnp.asarray(k, jnp.float32) * 0.0
